# baseline (speedup 1.0000x reference)
import functools

import jax
import jax.numpy as jnp
from jax import lax
from jax.experimental import pallas as pl
from jax.experimental.pallas import tpu as pltpu
from jax.experimental.pallas import tpu_sc as plsc

B = 16384
D = 128
DIM_OUT = 1024
NC, NS = 2, 16
NW = NC * NS
BPW = B // NW
CHUNK = 128
NCHUNK = BPW // CHUNK

_sc_mesh = plsc.VectorSubcoreMesh(core_axis_name="c", subcore_axis_name="s")


@functools.partial(
    pl.kernel,
    out_type=(
        jax.ShapeDtypeStruct((B // CHUNK, CHUNK, D), jnp.float32),
        jax.ShapeDtypeStruct((B // CHUNK, CHUNK, D), jnp.float32),
    ),
    mesh=_sc_mesh,
    scratch_types=[
        pltpu.VMEM((NCHUNK, CHUNK), jnp.int32),
        pltpu.VMEM((NCHUNK, CHUNK), jnp.int32),
        pltpu.VMEM((NCHUNK, CHUNK, D), jnp.float32),
        pltpu.SemaphoreType.DMA,
    ],
)
def _sc_gather(c0_hbm, c1_hbm, t0_hbm, t1_hbm, e0_hbm, e1_hbm,
               idx0_v, idx1_v, rows_v, sem):
    wid = lax.axis_index("s") * NC + lax.axis_index("c")
    cbase = wid * NCHUNK
    pltpu.sync_copy(c0_hbm.at[pl.ds(cbase, NCHUNK)], idx0_v)
    pltpu.sync_copy(c1_hbm.at[pl.ds(cbase, NCHUNK)], idx1_v)
    for j in range(NCHUNK):
        pltpu.async_copy(t0_hbm.at[idx0_v.at[j]], rows_v.at[j], sem)
    for j in range(NCHUNK):
        pltpu.make_async_copy(t0_hbm.at[idx0_v.at[j]], rows_v.at[j], sem).wait()
    pltpu.sync_copy(rows_v, e0_hbm.at[pl.ds(cbase, NCHUNK)])
    for j in range(NCHUNK):
        pltpu.async_copy(t1_hbm.at[idx1_v.at[j]], rows_v.at[j], sem)
    for j in range(NCHUNK):
        pltpu.make_async_copy(t1_hbm.at[idx1_v.at[j]], rows_v.at[j], sem).wait()
    pltpu.sync_copy(rows_v, e1_hbm.at[pl.ds(cbase, NCHUNK)])


BLK = 512


def _mlp_body(e0_ref, e1_ref, w0_ref, w1_ref, b_ref, o_ref):
    e0 = e0_ref[...].astype(jnp.bfloat16)
    e1 = e1_ref[...].astype(jnp.bfloat16)
    h = jnp.dot(e0, w0_ref[...], preferred_element_type=jnp.float32)
    h = h + jnp.dot(e1, w1_ref[...], preferred_element_type=jnp.float32)
    h = h + b_ref[...]
    o_ref[...] = h * jax.nn.sigmoid(h)


_mlp = pl.pallas_call(
    _mlp_body,
    grid=(B // BLK,),
    in_specs=[
        pl.BlockSpec((BLK, D), lambda i: (i, 0)),
        pl.BlockSpec((BLK, D), lambda i: (i, 0)),
        pl.BlockSpec((D, DIM_OUT), lambda i: (0, 0)),
        pl.BlockSpec((D, DIM_OUT), lambda i: (0, 0)),
        pl.BlockSpec((1, DIM_OUT), lambda i: (0, 0)),
    ],
    out_specs=pl.BlockSpec((BLK, DIM_OUT), lambda i: (i, 0)),
    out_shape=jax.ShapeDtypeStruct((B, DIM_OUT), jnp.float32),
)


@jax.jit
def kernel(cond, cond_embed0, cond_embed1, W, b):
    c0 = cond[:, 0].reshape(B // CHUNK, CHUNK)
    c1 = cond[:, 1].reshape(B // CHUNK, CHUNK)
    e0, e1 = _sc_gather(c0, c1, cond_embed0, cond_embed1)
    e0 = e0.reshape(B, D)
    e1 = e1.reshape(B, D)
    Wb = W.astype(jnp.bfloat16)
    return _mlp(e0, e1, Wb[:D], Wb[D:], b.reshape(1, DIM_OUT))


# speedup vs baseline: 1.1184x; 1.1184x over previous
import functools

import jax
import jax.numpy as jnp
from jax import lax
from jax.experimental import pallas as pl
from jax.experimental.pallas import tpu as pltpu
from jax.experimental.pallas import tpu_sc as plsc

B = 16384
D = 128
DIM_OUT = 1024
NC, NS = 2, 16
NW = NC * NS
BPW = B // NW
CHUNK = 128
NCHUNK = BPW // CHUNK

_sc_mesh = plsc.VectorSubcoreMesh(core_axis_name="c", subcore_axis_name="s")


@functools.partial(
    pl.kernel,
    out_type=(
        jax.ShapeDtypeStruct((B // CHUNK, CHUNK, D), jnp.float32),
        jax.ShapeDtypeStruct((B // CHUNK, CHUNK, D), jnp.float32),
    ),
    mesh=_sc_mesh,
    scratch_types=[
        pltpu.VMEM((NCHUNK, CHUNK), jnp.int32),
        pltpu.VMEM((NCHUNK, CHUNK), jnp.int32),
        pltpu.VMEM((NCHUNK, CHUNK, D), jnp.float32),
        pltpu.SemaphoreType.DMA,
    ],
)
def _sc_gather(c0_hbm, c1_hbm, t0_hbm, t1_hbm, e0_hbm, e1_hbm,
               idx0_v, idx1_v, rows_v, sem):
    wid = lax.axis_index("s") * NC + lax.axis_index("c")
    cbase = wid * NCHUNK
    pltpu.sync_copy(c0_hbm.at[pl.ds(cbase, NCHUNK)], idx0_v)
    pltpu.sync_copy(c1_hbm.at[pl.ds(cbase, NCHUNK)], idx1_v)
    for j in range(NCHUNK):
        pltpu.async_copy(t0_hbm.at[idx0_v.at[j]], rows_v.at[j], sem)
    for j in range(NCHUNK):
        pltpu.make_async_copy(t0_hbm.at[idx0_v.at[j]], rows_v.at[j], sem).wait()
    pltpu.sync_copy(rows_v, e0_hbm.at[pl.ds(cbase, NCHUNK)])
    for j in range(NCHUNK):
        pltpu.async_copy(t1_hbm.at[idx1_v.at[j]], rows_v.at[j], sem)
    for j in range(NCHUNK):
        pltpu.make_async_copy(t1_hbm.at[idx1_v.at[j]], rows_v.at[j], sem).wait()
    pltpu.sync_copy(rows_v, e1_hbm.at[pl.ds(cbase, NCHUNK)])


BLK = 1024


def _mlp_body(e0_ref, e1_ref, w0_ref, w1_ref, b_ref, o_ref):
    e0 = e0_ref[...].astype(jnp.bfloat16)
    e1 = e1_ref[...].astype(jnp.bfloat16)
    h = jnp.dot(e0, w0_ref[...], preferred_element_type=jnp.float32)
    h = h + jnp.dot(e1, w1_ref[...], preferred_element_type=jnp.float32)
    h = h + b_ref[...]
    o_ref[...] = h * jax.nn.sigmoid(h)


_mlp = pl.pallas_call(
    _mlp_body,
    grid=(B // BLK,),
    in_specs=[
        pl.BlockSpec((BLK, D), lambda i: (i, 0)),
        pl.BlockSpec((BLK, D), lambda i: (i, 0)),
        pl.BlockSpec((D, DIM_OUT), lambda i: (0, 0)),
        pl.BlockSpec((D, DIM_OUT), lambda i: (0, 0)),
        pl.BlockSpec((1, DIM_OUT), lambda i: (0, 0)),
    ],
    out_specs=pl.BlockSpec((BLK, DIM_OUT), lambda i: (i, 0)),
    out_shape=jax.ShapeDtypeStruct((B, DIM_OUT), jnp.float32),
)


@jax.jit
def kernel(cond, cond_embed0, cond_embed1, W, b):
    c0 = cond[:, 0].reshape(B // CHUNK, CHUNK)
    c1 = cond[:, 1].reshape(B // CHUNK, CHUNK)
    e0, e1 = _sc_gather(c0, c1, cond_embed0, cond_embed1)
    e0 = e0.reshape(B, D)
    e1 = e1.reshape(B, D)
    Wb = W.astype(jnp.bfloat16)
    return _mlp(e0, e1, Wb[:D], Wb[D:], b.reshape(1, DIM_OUT))


# speedup vs baseline: 1.1975x; 1.0708x over previous
import functools

import jax
import jax.numpy as jnp
from jax import lax
from jax.experimental import pallas as pl
from jax.experimental.pallas import tpu as pltpu
from jax.experimental.pallas import tpu_sc as plsc

B = 16384
D = 128
DIM_OUT = 1024
NC, NS = 2, 16
NW = NC * NS
BPW = B // NW
CHUNK = 128
NCHUNK = BPW // CHUNK

_sc_mesh = plsc.VectorSubcoreMesh(core_axis_name="c", subcore_axis_name="s")


@functools.partial(
    pl.kernel,
    out_type=(
        jax.ShapeDtypeStruct((B // CHUNK, CHUNK, D), jnp.float32),
        jax.ShapeDtypeStruct((B // CHUNK, CHUNK, D), jnp.float32),
    ),
    mesh=_sc_mesh,
    scratch_types=[
        pltpu.VMEM((NCHUNK, CHUNK), jnp.int32),
        pltpu.VMEM((NCHUNK, CHUNK), jnp.int32),
        pltpu.VMEM((NCHUNK, CHUNK, D), jnp.float32),
        pltpu.SemaphoreType.DMA,
    ],
)
def _sc_gather(c0_hbm, c1_hbm, t0_hbm, t1_hbm, e0_hbm, e1_hbm,
               idx0_v, idx1_v, rows_v, sem):
    wid = lax.axis_index("s") * NC + lax.axis_index("c")
    cbase = wid * NCHUNK
    pltpu.sync_copy(c0_hbm.at[pl.ds(cbase, NCHUNK)], idx0_v)
    pltpu.sync_copy(c1_hbm.at[pl.ds(cbase, NCHUNK)], idx1_v)
    for j in range(NCHUNK):
        pltpu.async_copy(t0_hbm.at[idx0_v.at[j]], rows_v.at[j], sem)
    for j in range(NCHUNK):
        pltpu.make_async_copy(t0_hbm.at[idx0_v.at[j]], rows_v.at[j], sem).wait()
    pltpu.sync_copy(rows_v, e0_hbm.at[pl.ds(cbase, NCHUNK)])
    for j in range(NCHUNK):
        pltpu.async_copy(t1_hbm.at[idx1_v.at[j]], rows_v.at[j], sem)
    for j in range(NCHUNK):
        pltpu.make_async_copy(t1_hbm.at[idx1_v.at[j]], rows_v.at[j], sem).wait()
    pltpu.sync_copy(rows_v, e1_hbm.at[pl.ds(cbase, NCHUNK)])


BLK = 2048


def _mlp_body(e0_ref, e1_ref, w0_ref, w1_ref, b_ref, o_ref):
    e0 = e0_ref[...].astype(jnp.bfloat16)
    e1 = e1_ref[...].astype(jnp.bfloat16)
    h = jnp.dot(e0, w0_ref[...], preferred_element_type=jnp.float32)
    h = h + jnp.dot(e1, w1_ref[...], preferred_element_type=jnp.float32)
    h = h + b_ref[...]
    o_ref[...] = h * jax.nn.sigmoid(h)


_mlp = pl.pallas_call(
    _mlp_body,
    grid=(B // BLK,),
    in_specs=[
        pl.BlockSpec((BLK, D), lambda i: (i, 0)),
        pl.BlockSpec((BLK, D), lambda i: (i, 0)),
        pl.BlockSpec((D, DIM_OUT), lambda i: (0, 0)),
        pl.BlockSpec((D, DIM_OUT), lambda i: (0, 0)),
        pl.BlockSpec((1, DIM_OUT), lambda i: (0, 0)),
    ],
    out_specs=pl.BlockSpec((BLK, DIM_OUT), lambda i: (i, 0)),
    out_shape=jax.ShapeDtypeStruct((B, DIM_OUT), jnp.float32),
)


@jax.jit
def kernel(cond, cond_embed0, cond_embed1, W, b):
    c0 = cond[:, 0].reshape(B // CHUNK, CHUNK)
    c1 = cond[:, 1].reshape(B // CHUNK, CHUNK)
    e0, e1 = _sc_gather(c0, c1, cond_embed0, cond_embed1)
    e0 = e0.reshape(B, D)
    e1 = e1.reshape(B, D)
    Wb = W.astype(jnp.bfloat16)
    return _mlp(e0, e1, Wb[:D], Wb[D:], b.reshape(1, DIM_OUT))
